# Initial kernel scaffold; baseline (speedup 1.0000x reference)
import jax
import jax.numpy as jnp
from jax.experimental import pallas as pl


def kernel(x_1, edge_index_1, x_2, edge_index_2, W1_1, b1_1, W2_1, b2_1, W3_1, b3_1, W4_1, b4_1, W1_2, b1_2, W2_2, b2_2, W3_2, b3_2, W4_2, b4_2):
    raise NotImplementedError("write your pallas kernel here")


# speedup vs baseline: 9.8646x; 9.8646x over previous
import functools

import jax
import jax.numpy as jnp
from jax import lax
from jax.experimental import pallas as pl
from jax.experimental.pallas import tpu as pltpu
from jax.experimental.pallas import tpu_sc as plsc

N = 10000
NP = 10240
F = 128
LOUT = 16
E = 320000
ET = E + N
NC, NS = 2, 16
NW = NC * NS
C = 128
K = -(-ET // (NW * C))
EPAD = NW * K * C
PAD_DST = N + 200
RPT = NP // NS
RB = 1280
GRID = NP // RB


def _sc_mesh():
    return plsc.VectorSubcoreMesh(
        core_axis_name="c", subcore_axis_name="s", num_cores=NC, num_subcores=NS
    )


def _make_agg_kernel():
    scratch = [
        pltpu.VMEM((K, C), jnp.int32),
        pltpu.VMEM((K, C), jnp.int32),
        pltpu.VMEM((C, F), jnp.float32),
        pltpu.VMEM_SHARED((NP, F), jnp.float32),
        pltpu.SemaphoreType.DMA,
    ]

    @functools.partial(
        pl.kernel,
        out_type=jax.ShapeDtypeStruct((NC, NP, F), jnp.float32),
        mesh=_sc_mesh(),
        scratch_types=scratch,
    )
    def agg(y_hbm, src_hbm, dst_hbm, z_hbm, out_hbm,
            src_v, dst_v, buf_v, acc_sh, sem):
        c = lax.axis_index("c")
        s = lax.axis_index("s")
        wid = c * NS + s
        pltpu.sync_copy(z_hbm, acc_sh.at[pl.ds(s * RPT, RPT)])
        pltpu.sync_copy(src_hbm.at[wid], src_v)
        pltpu.sync_copy(dst_hbm.at[wid], dst_v)
        plsc.subcore_barrier()

        def body(j, carry):
            pltpu.async_copy(y_hbm.at[src_v.at[j]], buf_v, sem).wait()
            pltpu.sync_copy(buf_v, acc_sh.at[dst_v.at[j]], add=True)
            return carry

        lax.fori_loop(0, K, body, 0)
        plsc.subcore_barrier()
        pltpu.sync_copy(acc_sh.at[pl.ds(s * RPT, RPT)],
                        out_hbm.at[c, pl.ds(s * RPT, RPT)])

    return agg


def _make_deg_kernel():
    scratch = [
        pltpu.VMEM((K, C), jnp.int32),
        pltpu.VMEM((NP,), jnp.float32),
    ]

    @functools.partial(
        pl.kernel,
        out_type=jax.ShapeDtypeStruct((NW, NP), jnp.float32),
        mesh=_sc_mesh(),
        scratch_types=scratch,
        compiler_params=pltpu.CompilerParams(needs_layout_passes=False),
    )
    def deg(dst_hbm, znp_hbm, out_hbm, dst_v, deg_v):
        c = lax.axis_index("c")
        s = lax.axis_index("s")
        wid = c * NS + s
        pltpu.sync_copy(znp_hbm, deg_v)
        pltpu.sync_copy(dst_hbm.at[wid], dst_v)
        ones16 = jnp.ones((16,), jnp.float32)

        def body(t, carry):
            j = t // (C // 16)
            i = t % (C // 16)
            idx = dst_v[j, pl.ds(i * 16, 16)]
            plsc.addupdate_scatter(deg_v, [idx], ones16)
            return carry

        lax.fori_loop(0, K * (C // 16), body, 0)
        pltpu.sync_copy(deg_v, out_hbm.at[wid])

    return deg


def _dinv_block(pdeg_blk):
    deg = jnp.sum(pdeg_blk, axis=0)
    return lax.rsqrt(jnp.maximum(deg, 1.0))[:, None]


def _tc_first(x, w, pdeg):
    def body(x_ref, w_ref, pdeg_ref, y_ref):
        dinv = _dinv_block(pdeg_ref[...])
        y_ref[...] = jnp.dot(x_ref[...] * dinv, w_ref[...],
                             preferred_element_type=jnp.float32)

    return pl.pallas_call(
        body,
        grid=(GRID,),
        in_specs=[
            pl.BlockSpec((RB, F), lambda i: (i, 0)),
            pl.BlockSpec((F, F), lambda i: (0, 0)),
            pl.BlockSpec((NW, RB), lambda i: (0, i)),
        ],
        out_specs=pl.BlockSpec((RB, F), lambda i: (i, 0)),
        out_shape=jax.ShapeDtypeStruct((NP, F), jnp.float32),
    )(x, w, pdeg)


def _tc_mid(p, pdeg, b2d, w):
    def body(p_ref, pdeg_ref, b_ref, w_ref, y_ref):
        dinv = _dinv_block(pdeg_ref[...])
        hcur = jnp.maximum((p_ref[0] + p_ref[1]) * dinv + b_ref[...], 0.0)
        y_ref[...] = jnp.dot(hcur * dinv, w_ref[...],
                             preferred_element_type=jnp.float32)

    return pl.pallas_call(
        body,
        grid=(GRID,),
        in_specs=[
            pl.BlockSpec((NC, RB, F), lambda i: (0, i, 0)),
            pl.BlockSpec((NW, RB), lambda i: (0, i)),
            pl.BlockSpec((1, F), lambda i: (0, 0)),
            pl.BlockSpec((F, F), lambda i: (0, 0)),
        ],
        out_specs=pl.BlockSpec((RB, F), lambda i: (i, 0)),
        out_shape=jax.ShapeDtypeStruct((NP, F), jnp.float32),
    )(p, pdeg, b2d, w)


def _tc_final(pa, pdega, ba2d, pb, pdegb, bb2d):
    def body(pa_ref, pdega_ref, ba_ref, pb_ref, pdegb_ref, bb_ref, o_ref):
        za = (pa_ref[0] + pa_ref[1]) * _dinv_block(pdega_ref[...]) + ba_ref[...]
        zb = (pb_ref[0] + pb_ref[1]) * _dinv_block(pdegb_ref[...]) + bb_ref[...]
        sa = 1.0 / (1.0 + jnp.exp(-za))
        sb = 1.0 / (1.0 + jnp.exp(-zb))
        o_ref[...] = ((sa + sb) * 0.5)[:, :LOUT]

    return pl.pallas_call(
        body,
        grid=(GRID,),
        in_specs=[
            pl.BlockSpec((NC, RB, F), lambda i: (0, i, 0)),
            pl.BlockSpec((NW, RB), lambda i: (0, i)),
            pl.BlockSpec((1, F), lambda i: (0, 0)),
            pl.BlockSpec((NC, RB, F), lambda i: (0, i, 0)),
            pl.BlockSpec((NW, RB), lambda i: (0, i)),
            pl.BlockSpec((1, F), lambda i: (0, 0)),
        ],
        out_specs=pl.BlockSpec((RB, LOUT), lambda i: (i, 0)),
        out_shape=jax.ShapeDtypeStruct((NP, LOUT), jnp.float32),
    )(pa, pdega, ba2d, pb, pdegb, bb2d)


def _edge_lists(edge_index):
    loop = jnp.arange(N, dtype=jnp.int32)
    src = jnp.concatenate([edge_index[0], loop,
                           jnp.zeros((EPAD - ET,), jnp.int32)])
    dst = jnp.concatenate([edge_index[1], loop,
                           jnp.full((EPAD - ET,), PAD_DST, jnp.int32)])
    return src.reshape(NW, K, C), dst.reshape(NW, K, C)


def _pad_cols(w, b):
    wp = jnp.zeros((F, F), jnp.float32).at[:, :LOUT].set(w)
    bp = jnp.zeros((1, F), jnp.float32).at[0, :LOUT].set(b)
    return wp, bp


def kernel(x_1, edge_index_1, x_2, edge_index_2,
           W1_1, b1_1, W2_1, b2_1, W3_1, b3_1, W4_1, b4_1,
           W1_2, b1_2, W2_2, b2_2, W3_2, b3_2, W4_2, b4_2):
    agg = _make_agg_kernel()
    deg_k = _make_deg_kernel()

    zf = jnp.zeros((RPT, F), jnp.float32)
    znp = jnp.zeros((NP,), jnp.float32)

    def tower(x, edge_index, params):
        (w1, b1), (w2, b2), (w3, b3), (w4, b4) = params
        src, dst = _edge_lists(edge_index)
        xp = jnp.zeros((NP, F), jnp.float32).at[:N].set(x)
        w4p, b4p = _pad_cols(w4, b4)
        pdeg = deg_k(dst, znp)
        y = _tc_first(xp, w1, pdeg)
        p = agg(y, src, dst, zf)
        y = _tc_mid(p, pdeg, b1.reshape(1, F), w2)
        p = agg(y, src, dst, zf)
        y = _tc_mid(p, pdeg, b2.reshape(1, F), w3)
        p = agg(y, src, dst, zf)
        y = _tc_mid(p, pdeg, b3.reshape(1, F), w4p)
        p = agg(y, src, dst, zf)
        return p, pdeg, b4p

    pa, pdega, ba = tower(x_1, edge_index_1,
                          [(W1_1, b1_1), (W2_1, b2_1), (W3_1, b3_1), (W4_1, b4_1)])
    x_2 = x_2 + 0.0 * pa[0, 0, 0]
    pb, pdegb, bb = tower(x_2, edge_index_2,
                          [(W1_2, b1_2), (W2_2, b2_2), (W3_2, b3_2), (W4_2, b4_2)])
    out = _tc_final(pa, pdega, ba, pb, pdegb, bb)
    return out[:N]
